# Initial kernel scaffold; baseline (speedup 1.0000x reference)
import jax
import jax.numpy as jnp
from jax.experimental import pallas as pl


def kernel(x, edge_index, batch, descriptor, params):
    raise NotImplementedError("write your pallas kernel here")


# speedup vs baseline: 1.9848x; 1.9848x over previous
import functools

import jax
import jax.numpy as jnp
from jax import lax
from jax.experimental import pallas as pl
from jax.experimental.pallas import tpu as pltpu
from jax.experimental.pallas import tpu_sc as plsc

F32 = jnp.float32
I32 = jnp.int32

NC = 2
NS = 16
K = 128


def _rup(x, m):
    return -(-x // m) * m


def _mesh():
    return plsc.VectorSubcoreMesh(core_axis_name="c", subcore_axis_name="s",
                                  num_cores=NC, num_subcores=NS)


def _sc_agg_kernel(n_pad, wp, nt_n, nchunk):
    ch_n = nt_n * NS
    n_per_core = nchunk // NC
    slab_len = (nt_n + 1) * wp

    @functools.partial(
        pl.kernel,
        out_type=jax.ShapeDtypeStruct((n_pad * wp,), F32),
        mesh=_mesh(),
        scratch_types=[
            pltpu.VMEM((K,), I32),
            pltpu.VMEM((K,), I32),
            pltpu.VMEM((K, wp), F32),
            pltpu.VMEM((280,), I32),
            pltpu.VMEM((slab_len,), F32),
        ],
    )
    def k(h_hbm, src_hbm, dl_hbm, bnd_hbm, out_hbm,
          idx_v, dl_v, rows_v, bnd_v, slab_v):
        c = lax.axis_index("c")
        s = lax.axis_index("s")
        pltpu.sync_copy(bnd_hbm, bnd_v)
        z16 = jnp.zeros((16,), F32)
        lanes = lax.iota(I32, 16)
        nzi = nt_n * wp // (16 * 8)

        def chunk_body(kc, _):
            gchunk = c * n_per_core + kc
            seg = gchunk * NS + s
            bv = bnd_v[pl.ds(seg, 16)]
            e0 = bv[0]
            e1 = bv[1]

            def zbody(i, _):
                for u in range(8):
                    off = pl.multiple_of((i * 8 + u) * 16, 16)
                    slab_v[pl.ds(off, 16)] = z16
                return 0

            lax.fori_loop(0, nzi, zbody, 0)

            e0a = e0 - lax.rem(e0, 8)
            nb = lax.div(e1 - e0a + (K - 1), K)

            def batch(j, _):
                base = pl.multiple_of(e0a + j * K, 8)
                pltpu.sync_copy(src_hbm.at[pl.ds(base, K)], idx_v)
                pltpu.sync_copy(dl_hbm.at[pl.ds(base, K)], dl_v)
                pltpu.sync_copy(h_hbm.at[idx_v], rows_v)

                def group(t, _):
                    d16 = dl_v[pl.ds(t * 16, 16)]
                    pos = base + t * 16 + lanes
                    keep = (pos >= e0) & (pos < e1)
                    d16 = jnp.where(keep, d16, nt_n)
                    for j2 in range(16):
                        roff = d16[j2] * wp
                        for t2 in range(wp // 16):
                            off = pl.multiple_of(roff + t2 * 16, 16)
                            plsc.addupdate(
                                slab_v.at[pl.ds(off, 16)],
                                rows_v[t * 16 + j2, pl.ds(t2 * 16, 16)])
                    return 0

                lax.fori_loop(0, K // 16, group, 0)
                return 0

            lax.fori_loop(0, nb, batch, 0)
            obase = pl.multiple_of((gchunk * ch_n + s * nt_n) * wp, 8)
            pltpu.sync_copy(slab_v.at[pl.ds(0, nt_n * wp)],
                            out_hbm.at[pl.ds(obase, nt_n * wp)])
            return 0

        lax.fori_loop(0, n_per_core, chunk_body, 0)

    return k


def _sc_pool_kernel(n_pad, wp, g):
    seg_tile = g // (NC * NS)
    slab_len = (seg_tile + 1) * wp

    @functools.partial(
        pl.kernel,
        out_type=jax.ShapeDtypeStruct((g * wp,), F32),
        mesh=_mesh(),
        scratch_types=[
            pltpu.VMEM((K,), I32),
            pltpu.VMEM((K, wp), F32),
            pltpu.VMEM((56,), I32),
            pltpu.VMEM((slab_len,), F32),
        ],
    )
    def k(h_hbm, bl_hbm, bnd_hbm, out_hbm, bl_v, rows_v, bnd_v, slab_v):
        c = lax.axis_index("c")
        s = lax.axis_index("s")
        pltpu.sync_copy(bnd_hbm, bnd_v)
        z16 = jnp.zeros((16,), F32)
        lanes = lax.iota(I32, 16)
        nzi = seg_tile * wp // (16 * 8)

        def zbody(i, _):
            for u in range(8):
                off = pl.multiple_of((i * 8 + u) * 16, 16)
                slab_v[pl.ds(off, 16)] = z16
            return 0

        lax.fori_loop(0, nzi, zbody, 0)

        tid = c * NS + s
        bv = bnd_v[pl.ds(tid, 16)]
        r0 = bv[0]
        r1 = bv[1]
        r0a = r0 - lax.rem(r0, 8)
        nb = lax.div(r1 - r0a + (K - 1), K)

        def batch(j, _):
            base = pl.multiple_of(r0a + j * K, 8)
            pltpu.sync_copy(bl_hbm.at[pl.ds(base, K)], bl_v)
            pltpu.sync_copy(h_hbm.at[pl.ds(base, K)], rows_v)

            def group(t, _):
                b16 = bl_v[pl.ds(t * 16, 16)]
                pos = base + t * 16 + lanes
                keep = (pos >= r0) & (pos < r1)
                b16 = jnp.where(keep, b16, seg_tile)
                for j2 in range(16):
                    roff = b16[j2] * wp
                    for t2 in range(wp // 16):
                        off = pl.multiple_of(roff + t2 * 16, 16)
                        plsc.addupdate(
                            slab_v.at[pl.ds(off, 16)],
                            rows_v[t * 16 + j2, pl.ds(t2 * 16, 16)])
                return 0

            lax.fori_loop(0, K // 16, group, 0)
            return 0

        lax.fori_loop(0, nb, batch, 0)
        obase = pl.multiple_of(tid * seg_tile * wp, 8)
        pltpu.sync_copy(slab_v.at[pl.ds(0, seg_tile * wp)],
                        out_hbm.at[pl.ds(obase, seg_tile * wp)])

    return k


_PREC = lax.Precision.HIGHEST


def _tc_mlp(n_pad, n_real, dip, dhp, br):
    grid = (n_pad // br,)

    def body(h_ref, a_ref, w1_ref, b1_ref, w2_ref, b2_ref, y_ref, st_ref):
        i = pl.program_id(0)
        t = h_ref[...] + a_ref[...]
        u = jnp.maximum(
            jnp.dot(t, w1_ref[...], preferred_element_type=F32,
                    precision=_PREC) + b1_ref[0:1, :], 0.0)
        y = jnp.dot(u, w2_ref[...], preferred_element_type=F32,
                    precision=_PREC) + b2_ref[0:1, :]
        y_ref[...] = y
        rid = lax.broadcasted_iota(I32, (br, 1), 0) + i * br
        ym = jnp.where(rid < n_real, y, 0.0)
        st = jnp.concatenate(
            [jnp.sum(ym, axis=0, keepdims=True),
             jnp.sum(ym * ym, axis=0, keepdims=True),
             jnp.zeros((6, dhp), F32)], axis=0)

        @pl.when(i == 0)
        def _():
            st_ref[...] = st

        @pl.when(i > 0)
        def _():
            st_ref[...] = st_ref[...] + st

    return pl.pallas_call(
        body,
        grid=grid,
        in_specs=[
            pl.BlockSpec((br, dip), lambda i: (i, 0)),
            pl.BlockSpec((br, dip), lambda i: (i, 0)),
            pl.BlockSpec((dip, dhp), lambda i: (0, 0)),
            pl.BlockSpec((8, dhp), lambda i: (0, 0)),
            pl.BlockSpec((dhp, dhp), lambda i: (0, 0)),
            pl.BlockSpec((8, dhp), lambda i: (0, 0)),
        ],
        out_specs=[
            pl.BlockSpec((br, dhp), lambda i: (i, 0)),
            pl.BlockSpec((8, dhp), lambda i: (0, 0)),
        ],
        out_shape=[
            jax.ShapeDtypeStruct((n_pad, dhp), F32),
            jax.ShapeDtypeStruct((8, dhp), F32),
        ],
    )


def _tc_norm(n_pad, n_real, dhp, br):
    grid = (n_pad // br,)
    inv_n = 1.0 / float(n_real)

    def body(y_ref, st_ref, gb_ref, o_ref):
        m = st_ref[0:1, :] * inv_n
        var = st_ref[1:2, :] * inv_n - m * m
        scale = gb_ref[0:1, :] * lax.rsqrt(var + 1e-5)
        o_ref[...] = (y_ref[...] - m) * scale + gb_ref[1:2, :]

    return pl.pallas_call(
        body,
        grid=grid,
        in_specs=[
            pl.BlockSpec((br, dhp), lambda i: (i, 0)),
            pl.BlockSpec((8, dhp), lambda i: (0, 0)),
            pl.BlockSpec((8, dhp), lambda i: (0, 0)),
        ],
        out_specs=pl.BlockSpec((br, dhp), lambda i: (i, 0)),
        out_shape=jax.ShapeDtypeStruct((n_pad, dhp), F32),
    )


def _tc_linear(m, kds, nout, relu):
    nx = len(kds)

    def body(*refs):
        xrefs = refs[:nx]
        wrefs = refs[nx:2 * nx]
        b_ref = refs[2 * nx]
        o_ref = refs[2 * nx + 1]
        acc = b_ref[0:1, :]
        for xr, wr in zip(xrefs, wrefs):
            acc = acc + jnp.dot(xr[...], wr[...], preferred_element_type=F32,
                                precision=_PREC)
        if relu:
            acc = jnp.maximum(acc, 0.0)
        o_ref[...] = acc

    return pl.pallas_call(
        body,
        in_specs=[pl.BlockSpec((m, kd), lambda: (0, 0)) for kd in kds]
        + [pl.BlockSpec((kd, nout), lambda: (0, 0)) for kd in kds]
        + [pl.BlockSpec((8, nout), lambda: (0, 0))],
        out_specs=pl.BlockSpec((m, nout), lambda: (0, 0)),
        out_shape=jax.ShapeDtypeStruct((m, nout), F32),
    )


def _pad2(a, r, c):
    return jnp.pad(a, ((0, r - a.shape[0]), (0, c - a.shape[1])))


def _bias8(b, c):
    return jnp.pad(b[None, :], ((0, 7), (0, c - b.shape[0])))


def kernel(x, edge_index, batch, descriptor, params):
    n, in_ch = x.shape
    e = edge_index.shape[1]
    g, n_desc = descriptor.shape

    sub = 256
    nchunk = 16
    nt_n = _rup(-(-n // sub), 8)
    n_pad = nt_n * sub
    br = 512 if n_pad % 512 == 0 else 128

    src, dst = edge_index[0], edge_index[1]
    dst_s, src_s = lax.sort((dst.astype(I32), src.astype(I32)), num_keys=1)
    e_pad = _rup(e + K, 1024)
    dl = (dst_s % nt_n).astype(I32)
    src_p = jnp.pad(src_s, (0, e_pad - e))
    dl_p = jnp.pad(dl, (0, e_pad - e))
    bnd = jnp.searchsorted(dst_s, (jnp.arange(sub + 1) * nt_n).astype(I32),
                           side="left").astype(I32)
    bnd = jnp.pad(bnd, (0, 280 - sub - 1))

    seg_tile = g // (NC * NS)
    rbnd = jnp.searchsorted(batch.astype(I32),
                            (jnp.arange(NC * NS + 1) * seg_tile).astype(I32),
                            side="left").astype(I32)
    rbnd = jnp.pad(rbnd, (0, 56 - NC * NS - 1))
    bl = (batch.astype(I32) % seg_tile).astype(I32)
    bl_p = jnp.pad(bl, (0, n_pad - n))

    convs = params["convs"]
    h = _pad2(x, n_pad, _rup(in_ch, 128))

    for (w1, b1, w2, b2, gam, bet) in convs:
        di, dh = w1.shape
        dip, dhp = _rup(di, 128), _rup(dh, 128)
        agg = _sc_agg_kernel(n_pad, dip, nt_n, nchunk)(
            h, src_p, dl_p, bnd).reshape(n_pad, dip)
        y, st = _tc_mlp(n_pad, n, dip, dhp, br)(
            h, agg, _pad2(w1, dip, dhp), _bias8(b1, dhp),
            _pad2(w2, dhp, dhp), _bias8(b2, dhp))
        gb = jnp.concatenate(
            [_bias8(gam, dhp)[0:1], _bias8(bet, dhp)[0:1],
             jnp.zeros((6, dhp), F32)], axis=0)
        h = _tc_norm(n_pad, n, dhp, br)(y, st, gb)

    wp = h.shape[1]
    pooled = _sc_pool_kernel(n_pad, wp, g)(h, bl_p, rbnd).reshape(g, wp)

    wg, bg = params["gf"]
    z = _tc_linear(g, [wp], wg.shape[1], True)(
        pooled, _pad2(wg, wp, wg.shape[1]), _bias8(bg, wg.shape[1]))

    fcs = params["fcs"]
    w1f, b1f = fcs[0]
    d_z = z.shape[1]
    d_dp = _rup(n_desc, 128)
    desc_p = _pad2(descriptor, g, d_dp)
    wa = w1f[:d_z]
    wb = _pad2(w1f[d_z:], d_dp, w1f.shape[1])
    z = _tc_linear(g, [d_z, d_dp], w1f.shape[1], True)(
        z, desc_p, wa, wb, _bias8(b1f, w1f.shape[1]))

    for i, (w, b) in enumerate(fcs[1:], start=1):
        din, dout = w.shape
        dop = _rup(dout, 128)
        last = i == len(fcs) - 1
        z = _tc_linear(g, [din], dop, not last)(
            z, _pad2(w, din, dop), _bias8(b, dop))

    return z[:, :1]


# speedup vs baseline: 3.1612x; 1.5927x over previous
import functools

import jax
import jax.numpy as jnp
from jax import lax
from jax.experimental import pallas as pl
from jax.experimental.pallas import tpu as pltpu
from jax.experimental.pallas import tpu_sc as plsc

F32 = jnp.float32
I32 = jnp.int32

NC = 2
NS = 16
K = 128


def _rup(x, m):
    return -(-x // m) * m


def _mesh():
    return plsc.VectorSubcoreMesh(core_axis_name="c", subcore_axis_name="s",
                                  num_cores=NC, num_subcores=NS)


def _sc_agg_kernel(n_pad, wp, nt_n, nchunk):
    ch_n = nt_n * NS
    n_per_core = nchunk // NC
    slab_len = (nt_n + 1) * wp

    @functools.partial(
        pl.kernel,
        out_type=jax.ShapeDtypeStruct((n_pad * wp,), F32),
        mesh=_mesh(),
        scratch_types=[
            pltpu.VMEM((K,), I32),
            pltpu.VMEM((K,), I32),
            pltpu.VMEM((K, wp), F32),
            pltpu.VMEM((280,), I32),
            pltpu.VMEM((slab_len,), F32),
        ],
    )
    def k(h_hbm, src_hbm, dl_hbm, bnd_hbm, out_hbm,
          idx_v, dl_v, rows_v, bnd_v, slab_v):
        c = lax.axis_index("c")
        s = lax.axis_index("s")
        pltpu.sync_copy(bnd_hbm, bnd_v)
        z16 = jnp.zeros((16,), F32)
        lanes = lax.iota(I32, 16)
        nzi = nt_n * wp // (16 * 8)
        nreg = wp // 16

        def flush(cur, acc):
            for t2 in range(nreg):
                off = pl.multiple_of(cur * wp + t2 * 16, 16)
                plsc.addupdate(slab_v.at[pl.ds(off, 16)], acc[t2])

        def chunk_body(kc, _):
            gchunk = c * n_per_core + kc
            seg = gchunk * NS + s
            bv = bnd_v[pl.ds(seg, 16)]
            e0 = bv[0]
            e1 = bv[1]

            def zbody(i, _):
                for u in range(8):
                    off = pl.multiple_of((i * 8 + u) * 16, 16)
                    slab_v[pl.ds(off, 16)] = z16
                return 0

            lax.fori_loop(0, nzi, zbody, 0)

            e0a = e0 - lax.rem(e0, 8)
            nb = lax.div(e1 - e0a + (K - 1), K)

            def batch(j, carry):
                base = pl.multiple_of(e0a + j * K, 8)
                pltpu.sync_copy(src_hbm.at[pl.ds(base, K)], idx_v)
                pltpu.sync_copy(dl_hbm.at[pl.ds(base, K)], dl_v)
                pltpu.sync_copy(h_hbm.at[idx_v], rows_v)

                def group(t, carry):
                    cur, acc = carry
                    d16 = dl_v[pl.ds(t * 16, 16)]
                    pos = base + t * 16 + lanes
                    keep = (pos >= e0) & (pos < e1)
                    d16 = jnp.where(keep, d16, nt_n)
                    for j2 in range(16):
                        d = d16[j2]
                        fl = d != cur

                        @pl.when(fl)
                        def _():
                            flush(cur, acc)

                        acc = tuple(
                            jnp.where(fl, 0.0, acc[t2])
                            + rows_v[t * 16 + j2, pl.ds(t2 * 16, 16)]
                            for t2 in range(nreg))
                        cur = d
                    return cur, acc

                return lax.fori_loop(0, K // 16, group, carry)

            acc0 = tuple(z16 for _ in range(nreg))
            cur, acc = lax.fori_loop(0, nb, batch, (jnp.int32(nt_n), acc0))
            flush(cur, acc)
            obase = pl.multiple_of((gchunk * ch_n + s * nt_n) * wp, 8)
            pltpu.sync_copy(slab_v.at[pl.ds(0, nt_n * wp)],
                            out_hbm.at[pl.ds(obase, nt_n * wp)])
            return 0

        lax.fori_loop(0, n_per_core, chunk_body, 0)

    return k


def _sc_pool_kernel(n_pad, wp, g):
    seg_tile = g // (NC * NS)
    slab_len = (seg_tile + 1) * wp

    @functools.partial(
        pl.kernel,
        out_type=jax.ShapeDtypeStruct((g * wp,), F32),
        mesh=_mesh(),
        scratch_types=[
            pltpu.VMEM((K,), I32),
            pltpu.VMEM((K, wp), F32),
            pltpu.VMEM((56,), I32),
            pltpu.VMEM((slab_len,), F32),
        ],
    )
    def k(h_hbm, bl_hbm, bnd_hbm, out_hbm, bl_v, rows_v, bnd_v, slab_v):
        c = lax.axis_index("c")
        s = lax.axis_index("s")
        pltpu.sync_copy(bnd_hbm, bnd_v)
        z16 = jnp.zeros((16,), F32)
        lanes = lax.iota(I32, 16)
        nzi = seg_tile * wp // (16 * 8)

        def zbody(i, _):
            for u in range(8):
                off = pl.multiple_of((i * 8 + u) * 16, 16)
                slab_v[pl.ds(off, 16)] = z16
            return 0

        lax.fori_loop(0, nzi, zbody, 0)

        tid = c * NS + s
        bv = bnd_v[pl.ds(tid, 16)]
        r0 = bv[0]
        r1 = bv[1]
        r0a = r0 - lax.rem(r0, 8)
        nb = lax.div(r1 - r0a + (K - 1), K)

        def batch(j, _):
            base = pl.multiple_of(r0a + j * K, 8)
            pltpu.sync_copy(bl_hbm.at[pl.ds(base, K)], bl_v)
            pltpu.sync_copy(h_hbm.at[pl.ds(base, K)], rows_v)

            def group(t, _):
                b16 = bl_v[pl.ds(t * 16, 16)]
                pos = base + t * 16 + lanes
                keep = (pos >= r0) & (pos < r1)
                b16 = jnp.where(keep, b16, seg_tile)
                for j2 in range(16):
                    roff = b16[j2] * wp
                    for t2 in range(wp // 16):
                        off = pl.multiple_of(roff + t2 * 16, 16)
                        plsc.addupdate(
                            slab_v.at[pl.ds(off, 16)],
                            rows_v[t * 16 + j2, pl.ds(t2 * 16, 16)])
                return 0

            lax.fori_loop(0, K // 16, group, 0)
            return 0

        lax.fori_loop(0, nb, batch, 0)
        obase = pl.multiple_of(tid * seg_tile * wp, 8)
        pltpu.sync_copy(slab_v.at[pl.ds(0, seg_tile * wp)],
                        out_hbm.at[pl.ds(obase, seg_tile * wp)])

    return k


_PREC = lax.Precision.DEFAULT


def _tc_mlp(n_pad, n_real, dip, dhp, br):
    grid = (n_pad // br,)

    def body(h_ref, a_ref, w1_ref, b1_ref, w2_ref, b2_ref, y_ref):
        t = h_ref[...] + a_ref[...]
        u = jnp.maximum(
            jnp.dot(t, w1_ref[...], preferred_element_type=F32,
                    precision=_PREC) + b1_ref[0:1, :], 0.0)
        y = jnp.dot(u, w2_ref[...], preferred_element_type=F32,
                    precision=_PREC) + b2_ref[0:1, :]
        y_ref[...] = y

    return pl.pallas_call(
        body,
        grid=grid,
        in_specs=[
            pl.BlockSpec((br, dip), lambda i: (i, 0)),
            pl.BlockSpec((br, dip), lambda i: (i, 0)),
            pl.BlockSpec((dip, dhp), lambda i: (0, 0)),
            pl.BlockSpec((8, dhp), lambda i: (0, 0)),
            pl.BlockSpec((dhp, dhp), lambda i: (0, 0)),
            pl.BlockSpec((8, dhp), lambda i: (0, 0)),
        ],
        out_specs=pl.BlockSpec((br, dhp), lambda i: (i, 0)),
        out_shape=jax.ShapeDtypeStruct((n_pad, dhp), F32),
    )


def _tc_norm(n_pad, dhp, br):
    grid = (n_pad // br,)

    def body(y_ref, mv_ref, gb_ref, o_ref):
        num = gb_ref[0:1, :] * (y_ref[...] - mv_ref[0:1, :])
        o_ref[...] = num / jnp.sqrt(mv_ref[1:2, :] + 1e-5) + gb_ref[1:2, :]

    return pl.pallas_call(
        body,
        grid=grid,
        in_specs=[
            pl.BlockSpec((br, dhp), lambda i: (i, 0)),
            pl.BlockSpec((8, dhp), lambda i: (0, 0)),
            pl.BlockSpec((8, dhp), lambda i: (0, 0)),
        ],
        out_specs=pl.BlockSpec((br, dhp), lambda i: (i, 0)),
        out_shape=jax.ShapeDtypeStruct((n_pad, dhp), F32),
    )


def _tc_linear(m, kds, nout, relu):
    nx = len(kds)

    def body(*refs):
        xrefs = refs[:nx]
        wrefs = refs[nx:2 * nx]
        b_ref = refs[2 * nx]
        o_ref = refs[2 * nx + 1]
        acc = b_ref[0:1, :]
        for xr, wr in zip(xrefs, wrefs):
            acc = acc + jnp.dot(xr[...], wr[...], preferred_element_type=F32,
                                precision=_PREC)
        if relu:
            acc = jnp.maximum(acc, 0.0)
        o_ref[...] = acc

    return pl.pallas_call(
        body,
        in_specs=[pl.BlockSpec((m, kd), lambda: (0, 0)) for kd in kds]
        + [pl.BlockSpec((kd, nout), lambda: (0, 0)) for kd in kds]
        + [pl.BlockSpec((8, nout), lambda: (0, 0))],
        out_specs=pl.BlockSpec((m, nout), lambda: (0, 0)),
        out_shape=jax.ShapeDtypeStruct((m, nout), F32),
    )


def _pad2(a, r, c):
    return jnp.pad(a, ((0, r - a.shape[0]), (0, c - a.shape[1])))


def _bias8(b, c):
    return jnp.pad(b[None, :], ((0, 7), (0, c - b.shape[0])))


def kernel(x, edge_index, batch, descriptor, params):
    n, in_ch = x.shape
    e = edge_index.shape[1]
    g, n_desc = descriptor.shape

    sub = 256
    nchunk = 16
    nt_n = _rup(-(-n // sub), 8)
    n_pad = nt_n * sub
    br = 512 if n_pad % 512 == 0 else 128

    src, dst = edge_index[0], edge_index[1]
    dst_s, src_s = lax.sort((dst.astype(I32), src.astype(I32)), num_keys=1)
    e_pad = _rup(e + K, 1024)
    dl = (dst_s % nt_n).astype(I32)
    src_p = jnp.pad(src_s, (0, e_pad - e))
    dl_p = jnp.pad(dl, (0, e_pad - e))
    bnd = jnp.searchsorted(dst_s, (jnp.arange(sub + 1) * nt_n).astype(I32),
                           side="left").astype(I32)
    bnd = jnp.pad(bnd, (0, 280 - sub - 1))

    seg_tile = g // (NC * NS)
    rbnd = jnp.searchsorted(batch.astype(I32),
                            (jnp.arange(NC * NS + 1) * seg_tile).astype(I32),
                            side="left").astype(I32)
    rbnd = jnp.pad(rbnd, (0, 56 - NC * NS - 1))
    bl = (batch.astype(I32) % seg_tile).astype(I32)
    bl_p = jnp.pad(bl, (0, n_pad - n))

    convs = params["convs"]
    h = _pad2(x, n_pad, _rup(in_ch, 128))

    for (w1, b1, w2, b2, gam, bet) in convs:
        di, dh = w1.shape
        dip, dhp = _rup(di, 128), _rup(dh, 128)
        agg = _sc_agg_kernel(n_pad, dip, nt_n, nchunk)(
            h, src_p, dl_p, bnd).reshape(n_pad, dip)
        y = _tc_mlp(n_pad, n, dip, dhp, br)(
            h, agg, _pad2(w1, dip, dhp), _bias8(b1, dhp),
            _pad2(w2, dhp, dhp), _bias8(b2, dhp))
        yr = y[:n, :dh]
        m = yr.mean(axis=0)
        v = ((yr - m) ** 2).mean(axis=0)
        m = jnp.pad(m, (0, dhp - dh))
        v = jnp.pad(v, (0, dhp - dh))
        gp = jnp.pad(gam, (0, dhp - dh))
        bp = jnp.pad(bet, (0, dhp - dh))
        h = gp * (y - m) / jnp.sqrt(v + 1e-5) + bp

    wp = h.shape[1]
    pooled = _sc_pool_kernel(n_pad, wp, g)(h, bl_p, rbnd).reshape(g, wp)

    wg, bg = params["gf"]
    z = _tc_linear(g, [wp], wg.shape[1], True)(
        pooled, _pad2(wg, wp, wg.shape[1]), _bias8(bg, wg.shape[1]))

    fcs = params["fcs"]
    w1f, b1f = fcs[0]
    d_z = z.shape[1]
    d_dp = _rup(n_desc, 128)
    desc_p = _pad2(descriptor, g, d_dp)
    wa = w1f[:d_z]
    wb = _pad2(w1f[d_z:], d_dp, w1f.shape[1])
    z = _tc_linear(g, [d_z, d_dp], w1f.shape[1], True)(
        z, desc_p, wa, wb, _bias8(b1f, w1f.shape[1]))

    for i, (w, b) in enumerate(fcs[1:], start=1):
        din, dout = w.shape
        dop = _rup(dout, 128)
        last = i == len(fcs) - 1
        z = _tc_linear(g, [din], dop, not last)(
            z, _pad2(w, din, dop), _bias8(b, dop))

    return z[:, :1]
